# baseline (speedup 1.0000x reference)
import functools

import jax
import jax.numpy as jnp
from jax import lax
from jax.experimental import pallas as pl
from jax.experimental.pallas import tpu as pltpu
from jax.experimental.pallas import tpu_sc as plsc

_ROWS = 262144
_DIM = 128
_NC = 2
_NS = 16
_NW = _NC * _NS
_ROWS_W = _ROWS // _NW
_CHUNK = 64
_NBUF = 2
_NITER = _ROWS_W // _CHUNK
_NGROUPS = _NITER // _NBUF


def _sc_copy_body(in_hbm, out_hbm, buf, in_sems, out_sems):
    cid = lax.axis_index("c")
    sid = lax.axis_index("s")
    wid = sid * _NC + cid
    base = wid * _ROWS_W

    def in_cp(row, b):
        return pltpu.make_async_copy(
            in_hbm.at[pl.ds(row, _CHUNK), :], buf.at[sid, b], in_sems.at[b]
        )

    def out_cp(row, b):
        return pltpu.make_async_copy(
            buf.at[sid, b], out_hbm.at[pl.ds(row, _CHUNK), :], out_sems.at[b]
        )

    for b in range(_NBUF):
        in_cp(base + b * _CHUNK, b).start()

    def group(g, carry):
        for b in range(_NBUF):
            row = base + (g * _NBUF + b) * _CHUNK
            in_cp(row, b).wait()
            out_cp(row, b).start()
            out_cp(row, b).wait()
            in_cp(row + _NBUF * _CHUNK, b).start()
        return carry

    lax.fori_loop(0, _NGROUPS - 1, group, 0)

    for b in range(_NBUF):
        row = base + ((_NGROUPS - 1) * _NBUF + b) * _CHUNK
        in_cp(row, b).wait()
        out_cp(row, b).start()
        out_cp(row, b).wait()


_sc_copy = functools.partial(
    pl.kernel,
    out_type=jax.ShapeDtypeStruct((_ROWS, _DIM), jnp.float32),
    mesh=plsc.VectorSubcoreMesh(core_axis_name="c", subcore_axis_name="s"),
    scratch_types=[
        pltpu.VMEM_SHARED((_NS, _NBUF, _CHUNK, _DIM), jnp.float32),
        pltpu.SemaphoreType.DMA((_NBUF,)),
        pltpu.SemaphoreType.DMA((_NBUF,)),
    ],
)(_sc_copy_body)


def kernel(queue):
    return _sc_copy(queue)


# speedup vs baseline: 1.7128x; 1.7128x over previous
import jax
import jax.numpy as jnp
from jax.experimental import pallas as pl
from jax.experimental.pallas import tpu as pltpu

_BLK = 16384


def _copy_body(in_ref, out_ref):
    out_ref[...] = in_ref[...]


def kernel(queue):
    rows, dim = queue.shape
    return pl.pallas_call(
        _copy_body,
        out_shape=jax.ShapeDtypeStruct(queue.shape, queue.dtype),
        grid=(rows // _BLK,),
        in_specs=[pl.BlockSpec((_BLK, dim), lambda i: (i, 0))],
        out_specs=pl.BlockSpec((_BLK, dim), lambda i: (i, 0)),
        compiler_params=pltpu.CompilerParams(
            dimension_semantics=("parallel",),
        ),
    )(queue)
